# baseline (speedup 1.0000x reference)
import jax
import jax.numpy as jnp
from jax.experimental import pallas as pl
from jax.experimental.pallas import tpu as pltpu

D = 1024
SEQ = 4096
BATCH = 4
RMAX = 1024
K = 8
KO = 3

_SIZES = ([512, 512] + [1024] * 3
          + [1024] * 4
          + [1024] * 4
          + [1024] * 3 + [512, 512])
_CHUNKS = []
_r0 = 0
for _nr in _SIZES:
    _CHUNKS.append((_r0, _nr))
    _r0 += _nr
assert _r0 == BATCH * SEQ

_PE_SIZES = [512, 512] + [1024] * 3
_PE = []
_s0 = 0
for _nr in _PE_SIZES:
    _PE.append((_s0, _nr))
    _s0 += _nr


def _pe_deps(row0, nr):
    s0 = row0 % SEQ
    return [q for q, (p0, pn) in enumerate(_PE)
            if p0 < s0 + nr and p0 + pn > s0]


def _body(sp_ref, x_any, pe_any, o_any, xbuf, pebuf, obuf, sx, spe, so):
    n = len(_CHUNKS)

    def x_copy(c):
        row0, nr = _CHUNKS[c]
        return pltpu.make_async_copy(
            x_any.at[pl.ds(row0, nr)],
            xbuf.at[c % K, pl.ds(0, nr)],
            sx.at[c % K],
        )

    def pe_copy(q):
        p0, pn = _PE[q]
        start = pl.multiple_of(sp_ref[0] + p0, 8)
        return pltpu.make_async_copy(
            pe_any.at[pl.ds(start, pn)],
            pebuf.at[pl.ds(p0, pn)],
            spe.at[q],
        )

    def o_copy(c):
        row0, nr = _CHUNKS[c]
        return pltpu.make_async_copy(
            obuf.at[c % KO, pl.ds(0, nr)],
            o_any.at[pl.ds(row0, nr)],
            so.at[c % KO],
        )

    x_copy(0).start()
    pe_copy(0).start()
    for i in range(1, K):
        x_copy(i).start()
    for q in range(1, len(_PE)):
        pe_copy(q).start()

    waited = set()
    for c in range(n):
        row0, nr = _CHUNKS[c]
        if c >= KO:
            o_copy(c - KO).wait()
        x_copy(c).wait()
        for q in _pe_deps(row0, nr):
            if q not in waited:
                pe_copy(q).wait()
                waited.add(q)
        s0 = row0 % SEQ
        obuf[c % KO, pl.ds(0, nr)] = (
            xbuf[c % K, pl.ds(0, nr)] + pebuf[pl.ds(s0, nr)]
        )
        o_copy(c).start()
        if c + K < n:
            x_copy(c + K).start()

    for c in range(n - KO, n):
        o_copy(c).wait()


@jax.jit
def _pe_add(sp, x, pos_embedding):
    batch, seq, d = x.shape
    xf = x.reshape(batch * seq, d)
    grid_spec = pltpu.PrefetchScalarGridSpec(
        num_scalar_prefetch=1,
        in_specs=[
            pl.BlockSpec(memory_space=pl.ANY),
            pl.BlockSpec(memory_space=pl.ANY),
        ],
        out_specs=pl.BlockSpec(memory_space=pl.ANY),
        scratch_shapes=[
            pltpu.VMEM((K, RMAX, d), jnp.float32),
            pltpu.VMEM((seq, d), jnp.float32),
            pltpu.VMEM((KO, RMAX, d), jnp.float32),
            pltpu.SemaphoreType.DMA((K,)),
            pltpu.SemaphoreType.DMA((len(_PE),)),
            pltpu.SemaphoreType.DMA((KO,)),
        ],
    )
    out = pl.pallas_call(
        _body,
        grid_spec=grid_spec,
        out_shape=jax.ShapeDtypeStruct(xf.shape, x.dtype),
        compiler_params=pltpu.CompilerParams(
            vmem_limit_bytes=62 * 1024 * 1024,
        ),
    )(sp, xf, pos_embedding)
    return out.reshape(x.shape)


def kernel(x, pos_embedding, start_pos):
    sp = jnp.atleast_1d(jnp.asarray(start_pos, dtype=jnp.int32))
    return _pe_add(sp, x, pos_embedding)


# speedup vs baseline: 1.0046x; 1.0046x over previous
import jax
import jax.numpy as jnp
from jax.experimental import pallas as pl
from jax.experimental.pallas import tpu as pltpu

D = 1024
SEQ = 4096
BATCH = 4
RMAX = 1024
K = 8
KO = 3

_SIZES = [1024] * 16
_CHUNKS = []
_r0 = 0
for _nr in _SIZES:
    _CHUNKS.append((_r0, _nr))
    _r0 += _nr
assert _r0 == BATCH * SEQ

_PE_SIZES = [1024] * 4
_PE = []
_s0 = 0
for _nr in _PE_SIZES:
    _PE.append((_s0, _nr))
    _s0 += _nr


def _pe_deps(row0, nr):
    s0 = row0 % SEQ
    return [q for q, (p0, pn) in enumerate(_PE)
            if p0 < s0 + nr and p0 + pn > s0]


def _body(sp_ref, x_any, pe_any, o_any, xbuf, pebuf, obuf, sx, spe, so):
    n = len(_CHUNKS)

    def x_copy(c):
        row0, nr = _CHUNKS[c]
        return pltpu.make_async_copy(
            x_any.at[pl.ds(row0, nr)],
            xbuf.at[c % K, pl.ds(0, nr)],
            sx.at[c % K],
        )

    def pe_copy(q):
        p0, pn = _PE[q]
        start = pl.multiple_of(sp_ref[0] + p0, 8)
        return pltpu.make_async_copy(
            pe_any.at[pl.ds(start, pn)],
            pebuf.at[pl.ds(p0, pn)],
            spe.at[q],
        )

    def o_copy(c):
        row0, nr = _CHUNKS[c]
        return pltpu.make_async_copy(
            obuf.at[c % KO, pl.ds(0, nr)],
            o_any.at[pl.ds(row0, nr)],
            so.at[c % KO],
        )

    x_copy(0).start()
    pe_copy(0).start()
    for i in range(1, K):
        x_copy(i).start()
    for q in range(1, len(_PE)):
        pe_copy(q).start()

    waited = set()
    for c in range(n):
        row0, nr = _CHUNKS[c]
        if c >= KO:
            o_copy(c - KO).wait()
        x_copy(c).wait()
        for q in _pe_deps(row0, nr):
            if q not in waited:
                pe_copy(q).wait()
                waited.add(q)
        s0 = row0 % SEQ
        obuf[c % KO, pl.ds(0, nr)] = (
            xbuf[c % K, pl.ds(0, nr)] + pebuf[pl.ds(s0, nr)]
        )
        o_copy(c).start()
        if c + K < n:
            x_copy(c + K).start()

    for c in range(n - KO, n):
        o_copy(c).wait()


@jax.jit
def _pe_add(sp, x, pos_embedding):
    batch, seq, d = x.shape
    xf = x.reshape(batch * seq, d)
    grid_spec = pltpu.PrefetchScalarGridSpec(
        num_scalar_prefetch=1,
        in_specs=[
            pl.BlockSpec(memory_space=pl.ANY),
            pl.BlockSpec(memory_space=pl.ANY),
        ],
        out_specs=pl.BlockSpec(memory_space=pl.ANY),
        scratch_shapes=[
            pltpu.VMEM((K, RMAX, d), jnp.float32),
            pltpu.VMEM((seq, d), jnp.float32),
            pltpu.VMEM((KO, RMAX, d), jnp.float32),
            pltpu.SemaphoreType.DMA((K,)),
            pltpu.SemaphoreType.DMA((len(_PE),)),
            pltpu.SemaphoreType.DMA((KO,)),
        ],
    )
    out = pl.pallas_call(
        _body,
        grid_spec=grid_spec,
        out_shape=jax.ShapeDtypeStruct(xf.shape, x.dtype),
        compiler_params=pltpu.CompilerParams(
            vmem_limit_bytes=62 * 1024 * 1024,
        ),
    )(sp, xf, pos_embedding)
    return out.reshape(x.shape)


def kernel(x, pos_embedding, start_pos):
    sp = jnp.atleast_1d(jnp.asarray(start_pos, dtype=jnp.int32))
    return _pe_add(sp, x, pos_embedding)
